# Initial kernel scaffold; baseline (speedup 1.0000x reference)
import jax
import jax.numpy as jnp
from jax.experimental import pallas as pl


def kernel(x, edge_index, batch, W1, b1, W2, b2, W3, b3, Wc1, bc1, Wc2, bc2):
    raise NotImplementedError("write your pallas kernel here")


# speedup vs baseline: 18.3094x; 18.3094x over previous
import functools

import jax
import jax.numpy as jnp
from jax import lax
from jax.experimental import pallas as pl
from jax.experimental.pallas import tpu as pltpu
from jax.experimental.pallas import tpu_sc as plsc

_N = 50000
_E = 800000
_G = 512
_H = 64

_NP = 53248
_EP = 802816
_ER = _EP // 128
_NR = _NP // 128
_GP = 520
_STRIPE = _NP // 16


def _mesh():
    return plsc.VectorSubcoreMesh(
        core_axis_name="c", subcore_axis_name="s", num_cores=2, num_subcores=16
    )


_SC_PARAMS = pltpu.CompilerParams(use_tc_tiling_on_sc=False)


def _sc_deg_body(dst2, ones, zeros, out, idx, ones_v, acc, sem):
    c = lax.axis_index("c")
    s = lax.axis_index("s")
    pltpu.sync_copy(zeros, acc.at[pl.ds(s * _STRIPE, _STRIPE)])
    pltpu.sync_copy(ones, ones_v)
    plsc.subcore_barrier()
    row0 = (c * 16 + s) * (_ER // 32)
    nb, bw = 14, 14

    def load(b, buf):
        pltpu.sync_copy(dst2.at[pl.ds(row0 + b * bw, bw)], idx.at[buf])

    def fire(buf):
        for j in range(bw):
            pltpu.async_copy(ones_v, acc.at[idx.at[buf].at[j]], sem, add=True)

    def drain(buf):
        for j in range(bw):
            pltpu.make_async_copy(
                ones_v, acc.at[idx.at[buf].at[j]], sem
            ).wait()

    load(0, 0)
    @pl.loop(0, nb)
    def _blk(b):
        buf = lax.rem(b, 2)
        nbuf = lax.rem(b + 1, 2)
        @pl.when(b > 0)
        def _w():
            drain(nbuf)
        @pl.when(b < nb - 1)
        def _l():
            load(b + 1, nbuf)
        fire(buf)
    drain((nb - 1) % 2)
    plsc.subcore_barrier()
    pltpu.sync_copy(
        acc.at[pl.ds(s * _STRIPE, _STRIPE)],
        out.at[c].at[pl.ds(s * _STRIPE, _STRIPE)],
    )


def _sc_deg(dst2, ones, zeros):
    return pl.kernel(
        _sc_deg_body,
        out_type=jax.ShapeDtypeStruct((2, _NP, 16), jnp.float32),
        mesh=_mesh(),
        compiler_params=_SC_PARAMS,
        scratch_types=[
            pltpu.VMEM((2, 14, 128), jnp.int32),
            pltpu.VMEM((128, 16), jnp.float32),
            pltpu.VMEM_SHARED((_NP, 16), jnp.float32),
            pltpu.SemaphoreType.DMA,
        ],
    )(dst2, ones, zeros)


def _sc_layer_body(hs, src2, dst2, zeros, out, idx_s, idx_d, rows, acc, sem, sem2):
    c = lax.axis_index("c")
    s = lax.axis_index("s")
    row0 = s * (_ER // 16)
    nb, bw = 28, 14
    for p in range(2):
        q = 2 * c + p
        pltpu.sync_copy(zeros, acc.at[pl.ds(s * _STRIPE, _STRIPE)])
        plsc.subcore_barrier()
        table = hs.at[q]

        def load_fire(b, buf):
            r0 = row0 + b * bw
            pltpu.sync_copy(src2.at[pl.ds(r0, bw)], idx_s.at[buf])
            pltpu.sync_copy(dst2.at[pl.ds(r0, bw)], idx_d.at[buf])
            for j in range(bw):
                pltpu.async_copy(
                    table.at[idx_s.at[buf].at[j]], rows.at[buf].at[j], sem
                )

        def wait_gathers(buf):
            for j in range(bw):
                pltpu.make_async_copy(
                    table.at[idx_s.at[buf].at[j]], rows.at[buf].at[j], sem
                ).wait()

        def fire_scatters(buf):
            for j in range(bw):
                pltpu.async_copy(
                    rows.at[buf].at[j], acc.at[idx_d.at[buf].at[j]], sem2, add=True
                )

        def wait_scatters(buf):
            for j in range(bw):
                pltpu.make_async_copy(
                    rows.at[buf].at[j], acc.at[idx_d.at[buf].at[j]], sem2
                ).wait()

        load_fire(0, 0)
        @pl.loop(0, nb)
        def _blk(b):
            buf = lax.rem(b, 2)
            nbuf = lax.rem(b + 1, 2)
            @pl.when(b > 0)
            def _ws():
                wait_scatters(nbuf)
            @pl.when(b < nb - 1)
            def _lf():
                load_fire(b + 1, nbuf)
            wait_gathers(buf)
            fire_scatters(buf)
        wait_scatters((nb - 1) % 2)
        plsc.subcore_barrier()
        pltpu.sync_copy(
            acc.at[pl.ds(s * _STRIPE, _STRIPE)],
            out.at[q].at[pl.ds(s * _STRIPE, _STRIPE)],
        )
        plsc.subcore_barrier()


def _sc_layer(hs, src2, dst2, zeros):
    return pl.kernel(
        _sc_layer_body,
        out_type=jax.ShapeDtypeStruct((4, _NP, 16), jnp.float32),
        mesh=_mesh(),
        compiler_params=_SC_PARAMS,
        scratch_types=[
            pltpu.VMEM((2, 14, 128), jnp.int32),
            pltpu.VMEM((2, 14, 128), jnp.int32),
            pltpu.VMEM((2, 14, 128, 16), jnp.float32),
            pltpu.VMEM_SHARED((_NP, 16), jnp.float32),
            pltpu.SemaphoreType.DMA,
            pltpu.SemaphoreType.DMA,
        ],
    )(hs, src2, dst2, zeros)


def _sc_pool_body(h3, bat2, zs, zc, ones, sums, cnts,
                  idx, rows_v, ones_v, segacc, cntacc, sem):
    c = lax.axis_index("c")
    s = lax.axis_index("s")

    @pl.when(s == 0)
    def _zero():
        pltpu.sync_copy(zs, segacc)
        pltpu.sync_copy(zc, cntacc)

    pltpu.sync_copy(ones, ones_v)
    rpt = _NR // 32
    row0 = c * (_NR // 2) + s * rpt
    pltpu.sync_copy(bat2.at[pl.ds(row0, rpt)], idx)
    plsc.subcore_barrier()
    @pl.loop(0, rpt)
    def _blk(b):
        pltpu.sync_copy(h3.at[pl.ds((row0 + b) * 128, 128)], rows_v)
        pltpu.sync_copy(rows_v, segacc.at[idx.at[b]], add=True)
        pltpu.sync_copy(ones_v, cntacc.at[idx.at[b]], add=True)
    plsc.subcore_barrier()

    @pl.when(s == 0)
    def _drain():
        pltpu.sync_copy(segacc, sums.at[c])
        pltpu.sync_copy(cntacc, cnts.at[c])


def _sc_pool(h3, bat2, zs, zc, ones):
    return pl.kernel(
        _sc_pool_body,
        out_type=[
            jax.ShapeDtypeStruct((2, _GP, 64), jnp.float32),
            jax.ShapeDtypeStruct((2, _GP, 16), jnp.float32),
        ],
        mesh=_mesh(),
        compiler_params=_SC_PARAMS,
        scratch_types=[
            pltpu.VMEM((_NR // 32, 128), jnp.int32),
            pltpu.VMEM((128, 64), jnp.float32),
            pltpu.VMEM((128, 16), jnp.float32),
            pltpu.VMEM_SHARED((_GP, 64), jnp.float32),
            pltpu.VMEM_SHARED((_GP, 16), jnp.float32),
            pltpu.SemaphoreType.DMA,
        ],
    )(h3, bat2, zs, zc, ones)


_R = 1024


def _dis_block(degp):
    deg = 1.0 + degp[0, :, :1] + degp[1, :, :1]
    return lax.rsqrt(deg)


def _tc_prep_body(degp, x, w1, hw_o, hs_o):
    dis = _dis_block(degp)
    hw = jnp.dot(x[...], w1[...], preferred_element_type=jnp.float32)
    hs = hw * dis
    hw_o[...] = hw
    for q in range(4):
        hs_o[q] = hs[:, 16 * q:16 * (q + 1)]


def _tc_prep(degp, x, w1):
    return pl.pallas_call(
        _tc_prep_body,
        grid=(_NP // _R,),
        in_specs=[
            pl.BlockSpec((2, _R, 16), lambda i: (0, i, 0)),
            pl.BlockSpec((_R, 64), lambda i: (i, 0)),
            pl.BlockSpec((64, 64), lambda i: (0, 0)),
        ],
        out_specs=[
            pl.BlockSpec((_R, 64), lambda i: (i, 0)),
            pl.BlockSpec((4, _R, 16), lambda i: (0, i, 0)),
        ],
        out_shape=[
            jax.ShapeDtypeStruct((_NP, 64), jnp.float32),
            jax.ShapeDtypeStruct((4, _NP, 16), jnp.float32),
        ],
    )(degp, x, w1)


def _tc_mid_body(degp, accp, hw, b, w, hw_o, hs_o):
    dis = _dis_block(degp)
    acccat = jnp.concatenate([accp[q] for q in range(4)], axis=1)
    h = jnp.maximum(dis * acccat + (dis * dis) * hw[...] + b[...], 0.0)
    hw2 = jnp.dot(h, w[...], preferred_element_type=jnp.float32)
    hs2 = hw2 * dis
    hw_o[...] = hw2
    for q in range(4):
        hs_o[q] = hs2[:, 16 * q:16 * (q + 1)]


def _tc_mid(degp, accp, hw, b, w):
    return pl.pallas_call(
        _tc_mid_body,
        grid=(_NP // _R,),
        in_specs=[
            pl.BlockSpec((2, _R, 16), lambda i: (0, i, 0)),
            pl.BlockSpec((4, _R, 16), lambda i: (0, i, 0)),
            pl.BlockSpec((_R, 64), lambda i: (i, 0)),
            pl.BlockSpec((1, 64), lambda i: (0, 0)),
            pl.BlockSpec((64, 64), lambda i: (0, 0)),
        ],
        out_specs=[
            pl.BlockSpec((_R, 64), lambda i: (i, 0)),
            pl.BlockSpec((4, _R, 16), lambda i: (0, i, 0)),
        ],
        out_shape=[
            jax.ShapeDtypeStruct((_NP, 64), jnp.float32),
            jax.ShapeDtypeStruct((4, _NP, 16), jnp.float32),
        ],
    )(degp, accp, hw, b, w)


def _tc_last_body(degp, accp, hw, b, h_o):
    dis = _dis_block(degp)
    acccat = jnp.concatenate([accp[q] for q in range(4)], axis=1)
    h_o[...] = jnp.maximum(dis * acccat + (dis * dis) * hw[...] + b[...], 0.0)


def _tc_last(degp, accp, hw, b):
    return pl.pallas_call(
        _tc_last_body,
        grid=(_NP // _R,),
        in_specs=[
            pl.BlockSpec((2, _R, 16), lambda i: (0, i, 0)),
            pl.BlockSpec((4, _R, 16), lambda i: (0, i, 0)),
            pl.BlockSpec((_R, 64), lambda i: (i, 0)),
            pl.BlockSpec((1, 64), lambda i: (0, 0)),
        ],
        out_specs=pl.BlockSpec((_R, 64), lambda i: (i, 0)),
        out_shape=jax.ShapeDtypeStruct((_NP, 64), jnp.float32),
    )(degp, accp, hw, b)


def _tc_head_body(sums, cnts, wc1, bc1, wc2, bc2, o):
    seg = sums[0, :_G, :] + sums[1, :_G, :]
    cnt = cnts[0, :_G, :1] + cnts[1, :_G, :1]
    pooled = seg / jnp.maximum(cnt, 1.0)
    z = jnp.maximum(
        jnp.dot(pooled, wc1[...], preferred_element_type=jnp.float32) + bc1[...],
        0.0,
    )
    o[...] = jnp.dot(z, wc2[...], preferred_element_type=jnp.float32) + bc2[...]


def _tc_head(sums, cnts, wc1, bc1, wc2, bc2):
    return pl.pallas_call(
        _tc_head_body,
        out_shape=jax.ShapeDtypeStruct((_G, 1), jnp.float32),
    )(sums, cnts, wc1, bc1, wc2, bc2)


def kernel(x, edge_index, batch, W1, b1, W2, b2, W3, b3, Wc1, bc1, Wc2, bc2):
    f32 = jnp.float32
    i32 = jnp.int32

    src = edge_index[0]
    dst = edge_index[1]
    pad_e = _EP - _E
    src2 = jnp.concatenate([src, jnp.zeros((pad_e,), i32)]).reshape(_ER, 128)
    dst2 = jnp.concatenate([dst, jnp.full((pad_e,), _N, i32)]).reshape(_ER, 128)
    bat2 = jnp.concatenate(
        [batch, jnp.full((_NP - _N,), _G, i32)]
    ).reshape(_NR, 128)
    x_p = jnp.concatenate([x, jnp.zeros((_NP - _N, 64), f32)], axis=0)

    ones16 = jnp.ones((128, 16), f32)
    z16 = jnp.zeros((_STRIPE, 16), f32)
    zgs = jnp.zeros((_GP, 64), f32)
    zgc = jnp.zeros((_GP, 16), f32)

    degp = _sc_deg(dst2, ones16, z16)

    hw1, hs1 = _tc_prep(degp, x_p, W1)
    acc1 = _sc_layer(hs1, src2, dst2, z16)
    hw2, hs2 = _tc_mid(degp, acc1, hw1, b1.reshape(1, 64), W2)
    acc2 = _sc_layer(hs2, src2, dst2, z16)
    hw3, hs3 = _tc_mid(degp, acc2, hw2, b2.reshape(1, 64), W3)
    acc3 = _sc_layer(hs3, src2, dst2, z16)
    h3 = _tc_last(degp, acc3, hw3, b3.reshape(1, 64))

    sums, cnts = _sc_pool(h3, bat2, zgs, zgc, ones16)
    return _tc_head(
        sums, cnts, Wc1, bc1.reshape(1, 32), Wc2, bc2.reshape(1, 1)
    )


# speedup vs baseline: 18.8782x; 1.0311x over previous
import functools

import jax
import jax.numpy as jnp
from jax import lax
from jax.experimental import pallas as pl
from jax.experimental.pallas import tpu as pltpu
from jax.experimental.pallas import tpu_sc as plsc

_N = 50000
_E = 800000
_G = 512
_H = 64

_NP = 53248
_EP = 802816
_ER = _EP // 128
_NR = _NP // 128
_GP = 520
_STRIPE = _NP // 16


def _mesh():
    return plsc.VectorSubcoreMesh(
        core_axis_name="c", subcore_axis_name="s", num_cores=2, num_subcores=16
    )


_SC_PARAMS = pltpu.CompilerParams(use_tc_tiling_on_sc=False)


def _sc_deg_body(dst2, ones, zeros, out, idx, ones_v, acc, sem):
    c = lax.axis_index("c")
    s = lax.axis_index("s")
    pltpu.sync_copy(zeros, acc.at[pl.ds(s * _STRIPE, _STRIPE)])
    pltpu.sync_copy(ones, ones_v)
    plsc.subcore_barrier()
    row0 = (c * 16 + s) * (_ER // 32)
    nb, bw = 14, 14

    def load(b, buf):
        pltpu.sync_copy(dst2.at[pl.ds(row0 + b * bw, bw)], idx.at[buf])

    def fire(buf):
        for j in range(bw):
            pltpu.async_copy(ones_v, acc.at[idx.at[buf].at[j]], sem, add=True)

    def drain(buf):
        for j in range(bw):
            pltpu.make_async_copy(
                ones_v, acc.at[idx.at[buf].at[j]], sem
            ).wait()

    load(0, 0)
    @pl.loop(0, nb)
    def _blk(b):
        buf = lax.rem(b, 2)
        nbuf = lax.rem(b + 1, 2)
        @pl.when(b > 0)
        def _w():
            drain(nbuf)
        @pl.when(b < nb - 1)
        def _l():
            load(b + 1, nbuf)
        fire(buf)
    drain((nb - 1) % 2)
    plsc.subcore_barrier()
    pltpu.sync_copy(
        acc.at[pl.ds(s * _STRIPE, _STRIPE)],
        out.at[c].at[pl.ds(s * _STRIPE, _STRIPE)],
    )


def _sc_deg(dst2, ones, zeros):
    return pl.kernel(
        _sc_deg_body,
        out_type=jax.ShapeDtypeStruct((2, _NP, 16), jnp.float32),
        mesh=_mesh(),
        compiler_params=_SC_PARAMS,
        scratch_types=[
            pltpu.VMEM((2, 14, 128), jnp.int32),
            pltpu.VMEM((128, 16), jnp.float32),
            pltpu.VMEM_SHARED((_NP, 16), jnp.float32),
            pltpu.SemaphoreType.DMA,
        ],
    )(dst2, ones, zeros)


def _sc_layer_body(hs, src2, dst2, zeros, out, idx_s, idx_d, rows, acc, sem, sem2):
    c = lax.axis_index("c")
    s = lax.axis_index("s")
    row0 = s * (_ER // 16)
    nb, bw = 28, 14
    for p in range(2):
        q = 2 * c + p
        pltpu.sync_copy(zeros, acc.at[pl.ds(s * _STRIPE, _STRIPE)])
        plsc.subcore_barrier()
        table = hs.at[q]

        def load_fire(b, buf):
            r0 = row0 + b * bw
            pltpu.sync_copy(src2.at[pl.ds(r0, bw)], idx_s.at[buf])
            pltpu.sync_copy(dst2.at[pl.ds(r0, bw)], idx_d.at[buf])
            for j in range(bw):
                pltpu.async_copy(
                    table.at[idx_s.at[buf].at[j]], rows.at[buf].at[j], sem
                )

        def wait_gathers(buf):
            for j in range(bw):
                pltpu.make_async_copy(
                    table.at[idx_s.at[buf].at[j]], rows.at[buf].at[j], sem
                ).wait()

        def fire_scatters(buf):
            for j in range(bw):
                pltpu.async_copy(
                    rows.at[buf].at[j], acc.at[idx_d.at[buf].at[j]], sem2, add=True
                )

        def wait_scatters(buf):
            for j in range(bw):
                pltpu.make_async_copy(
                    rows.at[buf].at[j], acc.at[idx_d.at[buf].at[j]], sem2
                ).wait()

        load_fire(0, 0)
        @pl.loop(0, nb)
        def _blk(b):
            buf = lax.rem(b, 2)
            nbuf = lax.rem(b + 1, 2)
            @pl.when(b > 0)
            def _ws():
                wait_scatters(nbuf)
            @pl.when(b < nb - 1)
            def _lf():
                load_fire(b + 1, nbuf)
            wait_gathers(buf)
            fire_scatters(buf)
        wait_scatters((nb - 1) % 2)
        plsc.subcore_barrier()
        pltpu.sync_copy(
            acc.at[pl.ds(s * _STRIPE, _STRIPE)],
            out.at[q].at[pl.ds(s * _STRIPE, _STRIPE)],
        )
        plsc.subcore_barrier()


def _sc_layer(hs, src2, dst2, zeros):
    return pl.kernel(
        _sc_layer_body,
        out_type=jax.ShapeDtypeStruct((4, _NP, 16), jnp.float32),
        mesh=_mesh(),
        compiler_params=_SC_PARAMS,
        scratch_types=[
            pltpu.VMEM((2, 14, 128), jnp.int32),
            pltpu.VMEM((2, 14, 128), jnp.int32),
            pltpu.VMEM((2, 14, 128, 16), jnp.float32),
            pltpu.VMEM_SHARED((_NP, 16), jnp.float32),
            pltpu.SemaphoreType.DMA,
            pltpu.SemaphoreType.DMA,
        ],
    )(hs, src2, dst2, zeros)


def _sc_pool_body(h3, bat2, zs, zc, ones, sums, cnts,
                  idx, rows_v, ones_v, segacc, cntacc, sem):
    c = lax.axis_index("c")
    s = lax.axis_index("s")

    @pl.when(s == 0)
    def _zero():
        pltpu.sync_copy(zs, segacc)
        pltpu.sync_copy(zc, cntacc)

    pltpu.sync_copy(ones, ones_v)
    rpt = _NR // 32
    row0 = c * (_NR // 2) + s * rpt
    pltpu.sync_copy(bat2.at[pl.ds(row0, rpt)], idx)
    plsc.subcore_barrier()
    @pl.loop(0, rpt)
    def _blk(b):
        pltpu.sync_copy(h3.at[pl.ds((row0 + b) * 128, 128)], rows_v)
        pltpu.sync_copy(rows_v, segacc.at[idx.at[b]], add=True)
        pltpu.sync_copy(ones_v, cntacc.at[idx.at[b]], add=True)
    plsc.subcore_barrier()

    @pl.when(s == 0)
    def _drain():
        pltpu.sync_copy(segacc, sums.at[c])
        pltpu.sync_copy(cntacc, cnts.at[c])


def _sc_pool(h3, bat2, zs, zc, ones):
    return pl.kernel(
        _sc_pool_body,
        out_type=[
            jax.ShapeDtypeStruct((2, _GP, 64), jnp.float32),
            jax.ShapeDtypeStruct((2, _GP, 16), jnp.float32),
        ],
        mesh=_mesh(),
        compiler_params=_SC_PARAMS,
        scratch_types=[
            pltpu.VMEM((_NR // 32, 128), jnp.int32),
            pltpu.VMEM((128, 64), jnp.float32),
            pltpu.VMEM((128, 16), jnp.float32),
            pltpu.VMEM_SHARED((_GP, 64), jnp.float32),
            pltpu.VMEM_SHARED((_GP, 16), jnp.float32),
            pltpu.SemaphoreType.DMA,
        ],
    )(h3, bat2, zs, zc, ones)


_R = 4096


def _dis_block(degp):
    deg = 1.0 + degp[0, :, :1] + degp[1, :, :1]
    return lax.rsqrt(deg)


def _tc_prep_body(degp, x, w1, hw_o, hs_o):
    dis = _dis_block(degp)
    hw = jnp.dot(x[...], w1[...], preferred_element_type=jnp.float32)
    hs = hw * dis
    hw_o[...] = hw
    for q in range(4):
        hs_o[q] = hs[:, 16 * q:16 * (q + 1)]


def _tc_prep(degp, x, w1):
    return pl.pallas_call(
        _tc_prep_body,
        grid=(_NP // _R,),
        in_specs=[
            pl.BlockSpec((2, _R, 16), lambda i: (0, i, 0)),
            pl.BlockSpec((_R, 64), lambda i: (i, 0)),
            pl.BlockSpec((64, 64), lambda i: (0, 0)),
        ],
        out_specs=[
            pl.BlockSpec((_R, 64), lambda i: (i, 0)),
            pl.BlockSpec((4, _R, 16), lambda i: (0, i, 0)),
        ],
        out_shape=[
            jax.ShapeDtypeStruct((_NP, 64), jnp.float32),
            jax.ShapeDtypeStruct((4, _NP, 16), jnp.float32),
        ],
    )(degp, x, w1)


def _tc_mid_body(degp, accp, hw, b, w, hw_o, hs_o):
    dis = _dis_block(degp)
    acccat = jnp.concatenate([accp[q] for q in range(4)], axis=1)
    h = jnp.maximum(dis * acccat + (dis * dis) * hw[...] + b[...], 0.0)
    hw2 = jnp.dot(h, w[...], preferred_element_type=jnp.float32)
    hs2 = hw2 * dis
    hw_o[...] = hw2
    for q in range(4):
        hs_o[q] = hs2[:, 16 * q:16 * (q + 1)]


def _tc_mid(degp, accp, hw, b, w):
    return pl.pallas_call(
        _tc_mid_body,
        grid=(_NP // _R,),
        in_specs=[
            pl.BlockSpec((2, _R, 16), lambda i: (0, i, 0)),
            pl.BlockSpec((4, _R, 16), lambda i: (0, i, 0)),
            pl.BlockSpec((_R, 64), lambda i: (i, 0)),
            pl.BlockSpec((1, 64), lambda i: (0, 0)),
            pl.BlockSpec((64, 64), lambda i: (0, 0)),
        ],
        out_specs=[
            pl.BlockSpec((_R, 64), lambda i: (i, 0)),
            pl.BlockSpec((4, _R, 16), lambda i: (0, i, 0)),
        ],
        out_shape=[
            jax.ShapeDtypeStruct((_NP, 64), jnp.float32),
            jax.ShapeDtypeStruct((4, _NP, 16), jnp.float32),
        ],
    )(degp, accp, hw, b, w)


def _tc_last_body(degp, accp, hw, b, h_o):
    dis = _dis_block(degp)
    acccat = jnp.concatenate([accp[q] for q in range(4)], axis=1)
    h_o[...] = jnp.maximum(dis * acccat + (dis * dis) * hw[...] + b[...], 0.0)


def _tc_last(degp, accp, hw, b):
    return pl.pallas_call(
        _tc_last_body,
        grid=(_NP // _R,),
        in_specs=[
            pl.BlockSpec((2, _R, 16), lambda i: (0, i, 0)),
            pl.BlockSpec((4, _R, 16), lambda i: (0, i, 0)),
            pl.BlockSpec((_R, 64), lambda i: (i, 0)),
            pl.BlockSpec((1, 64), lambda i: (0, 0)),
        ],
        out_specs=pl.BlockSpec((_R, 64), lambda i: (i, 0)),
        out_shape=jax.ShapeDtypeStruct((_NP, 64), jnp.float32),
    )(degp, accp, hw, b)


def _tc_head_body(sums, cnts, wc1, bc1, wc2, bc2, o):
    seg = sums[0, :_G, :] + sums[1, :_G, :]
    cnt = cnts[0, :_G, :1] + cnts[1, :_G, :1]
    pooled = seg / jnp.maximum(cnt, 1.0)
    z = jnp.maximum(
        jnp.dot(pooled, wc1[...], preferred_element_type=jnp.float32) + bc1[...],
        0.0,
    )
    o[...] = jnp.dot(z, wc2[...], preferred_element_type=jnp.float32) + bc2[...]


def _tc_head(sums, cnts, wc1, bc1, wc2, bc2):
    return pl.pallas_call(
        _tc_head_body,
        out_shape=jax.ShapeDtypeStruct((_G, 1), jnp.float32),
    )(sums, cnts, wc1, bc1, wc2, bc2)


def kernel(x, edge_index, batch, W1, b1, W2, b2, W3, b3, Wc1, bc1, Wc2, bc2):
    f32 = jnp.float32
    i32 = jnp.int32

    src = edge_index[0]
    dst = edge_index[1]
    pad_e = _EP - _E
    src2 = jnp.concatenate([src, jnp.zeros((pad_e,), i32)]).reshape(_ER, 128)
    dst2 = jnp.concatenate([dst, jnp.full((pad_e,), _N, i32)]).reshape(_ER, 128)
    bat2 = jnp.concatenate(
        [batch, jnp.full((_NP - _N,), _G, i32)]
    ).reshape(_NR, 128)
    x_p = jnp.concatenate([x, jnp.zeros((_NP - _N, 64), f32)], axis=0)

    ones16 = jnp.ones((128, 16), f32)
    z16 = jnp.zeros((_STRIPE, 16), f32)
    zgs = jnp.zeros((_GP, 64), f32)
    zgc = jnp.zeros((_GP, 16), f32)

    degp = _sc_deg(dst2, ones16, z16)

    hw1, hs1 = _tc_prep(degp, x_p, W1)
    acc1 = _sc_layer(hs1, src2, dst2, z16)
    hw2, hs2 = _tc_mid(degp, acc1, hw1, b1.reshape(1, 64), W2)
    acc2 = _sc_layer(hs2, src2, dst2, z16)
    hw3, hs3 = _tc_mid(degp, acc2, hw2, b2.reshape(1, 64), W3)
    acc3 = _sc_layer(hs3, src2, dst2, z16)
    h3 = _tc_last(degp, acc3, hw3, b3.reshape(1, 64))

    sums, cnts = _sc_pool(h3, bat2, zgs, zgc, ones16)
    return _tc_head(
        sums, cnts, Wc1, bc1.reshape(1, 32), Wc2, bc2.reshape(1, 1)
    )


# speedup vs baseline: 20.4219x; 1.0818x over previous
import functools

import jax
import jax.numpy as jnp
from jax import lax
from jax.experimental import pallas as pl
from jax.experimental.pallas import tpu as pltpu
from jax.experimental.pallas import tpu_sc as plsc

_N = 50000
_E = 800000
_G = 512
_H = 64

_NP = 53248
_EP = 802816
_ER = _EP // 128
_NR = _NP // 128
_GP = 520
_STRIPE = _NP // 16


def _mesh():
    return plsc.VectorSubcoreMesh(
        core_axis_name="c", subcore_axis_name="s", num_cores=2, num_subcores=16
    )


_SC_PARAMS = pltpu.CompilerParams(use_tc_tiling_on_sc=False)


def _sc_deg_body(dst2, ones, zeros, out, idx, ones_v, acc, sem):
    c = lax.axis_index("c")
    s = lax.axis_index("s")
    pltpu.sync_copy(zeros, acc.at[pl.ds(s * _STRIPE, _STRIPE)])
    pltpu.sync_copy(ones, ones_v)
    plsc.subcore_barrier()
    row0 = (c * 16 + s) * (_ER // 32)
    nb, bw = 14, 14

    def load(b, buf):
        pltpu.sync_copy(dst2.at[pl.ds(row0 + b * bw, bw)], idx.at[buf])

    def fire(buf):
        for j in range(bw):
            pltpu.async_copy(ones_v, acc.at[idx.at[buf].at[j]], sem, add=True)

    def drain(buf):
        for j in range(bw):
            pltpu.make_async_copy(
                ones_v, acc.at[idx.at[buf].at[j]], sem
            ).wait()

    load(0, 0)
    @pl.loop(0, nb)
    def _blk(b):
        buf = lax.rem(b, 2)
        nbuf = lax.rem(b + 1, 2)
        @pl.when(b > 0)
        def _w():
            drain(nbuf)
        @pl.when(b < nb - 1)
        def _l():
            load(b + 1, nbuf)
        fire(buf)
    drain((nb - 1) % 2)
    plsc.subcore_barrier()
    pltpu.sync_copy(
        acc.at[pl.ds(s * _STRIPE, _STRIPE)],
        out.at[c].at[pl.ds(s * _STRIPE, _STRIPE)],
    )


def _sc_deg(dst2, ones, zeros):
    return pl.kernel(
        _sc_deg_body,
        out_type=jax.ShapeDtypeStruct((2, _NP, 16), jnp.float32),
        mesh=_mesh(),
        compiler_params=_SC_PARAMS,
        scratch_types=[
            pltpu.VMEM((2, 14, 128), jnp.int32),
            pltpu.VMEM((128, 16), jnp.float32),
            pltpu.VMEM_SHARED((_NP, 16), jnp.float32),
            pltpu.SemaphoreType.DMA,
        ],
    )(dst2, ones, zeros)


def _sc_layer_body(hs, src2, dst2, zeros, out, idx_s, idx_d, rows, acc, sem, sem2):
    c = lax.axis_index("c")
    s = lax.axis_index("s")
    row0 = s * (_ER // 16)
    nb, bw = 28, 14
    for p in range(2):
        q = 2 * c + p
        pltpu.sync_copy(zeros, acc.at[pl.ds(s * _STRIPE, _STRIPE)])
        plsc.subcore_barrier()
        table = hs.at[q]

        def load_fire(b, buf):
            r0 = row0 + b * bw
            pltpu.sync_copy(src2.at[pl.ds(r0, bw)], idx_s.at[buf])
            pltpu.sync_copy(dst2.at[pl.ds(r0, bw)], idx_d.at[buf])
            for j in range(bw):
                pltpu.async_copy(
                    table.at[idx_s.at[buf].at[j]], rows.at[buf].at[j], sem
                )

        def wait_gathers(buf):
            for j in range(bw):
                pltpu.make_async_copy(
                    table.at[idx_s.at[buf].at[j]], rows.at[buf].at[j], sem
                ).wait()

        def fire_scatters(buf):
            for j in range(bw):
                pltpu.async_copy(
                    rows.at[buf].at[j], acc.at[idx_d.at[buf].at[j]], sem2, add=True
                )

        def wait_scatters(buf):
            for j in range(bw):
                pltpu.make_async_copy(
                    rows.at[buf].at[j], acc.at[idx_d.at[buf].at[j]], sem2
                ).wait()

        load_fire(0, 0)
        @pl.loop(0, nb)
        def _blk(b):
            buf = lax.rem(b, 2)
            nbuf = lax.rem(b + 1, 2)
            @pl.when(b > 0)
            def _ws():
                wait_scatters(nbuf)
            @pl.when(b < nb - 1)
            def _lf():
                load_fire(b + 1, nbuf)
            wait_gathers(buf)
            fire_scatters(buf)
        wait_scatters((nb - 1) % 2)
        plsc.subcore_barrier()
        pltpu.sync_copy(
            acc.at[pl.ds(s * _STRIPE, _STRIPE)],
            out.at[pl.ds(s * _STRIPE, _STRIPE), pl.ds(16 * q, 16)],
        )
        plsc.subcore_barrier()


def _sc_layer(hs, src2, dst2, zeros):
    return pl.kernel(
        _sc_layer_body,
        out_type=jax.ShapeDtypeStruct((_NP, 64), jnp.float32),
        mesh=_mesh(),
        compiler_params=_SC_PARAMS,
        scratch_types=[
            pltpu.VMEM((2, 14, 128), jnp.int32),
            pltpu.VMEM((2, 14, 128), jnp.int32),
            pltpu.VMEM((2, 14, 128, 16), jnp.float32),
            pltpu.VMEM_SHARED((_NP, 16), jnp.float32),
            pltpu.SemaphoreType.DMA,
            pltpu.SemaphoreType.DMA,
        ],
    )(hs, src2, dst2, zeros)


def _sc_pool_body(h3, bat2, zs, zc, ones, sums, cnts,
                  idx, rows_v, ones_v, segacc, cntacc, sem):
    c = lax.axis_index("c")
    s = lax.axis_index("s")

    @pl.when(s == 0)
    def _zero():
        pltpu.sync_copy(zs, segacc)
        pltpu.sync_copy(zc, cntacc)

    pltpu.sync_copy(ones, ones_v)
    rpt = _NR // 32
    row0 = c * (_NR // 2) + s * rpt
    pltpu.sync_copy(bat2.at[pl.ds(row0, rpt)], idx)
    plsc.subcore_barrier()
    @pl.loop(0, rpt)
    def _blk(b):
        pltpu.sync_copy(h3.at[pl.ds((row0 + b) * 128, 128)], rows_v)
        pltpu.sync_copy(rows_v, segacc.at[idx.at[b]], add=True)
        pltpu.sync_copy(ones_v, cntacc.at[idx.at[b]], add=True)
    plsc.subcore_barrier()

    @pl.when(s == 0)
    def _drain():
        pltpu.sync_copy(segacc, sums.at[c])
        pltpu.sync_copy(cntacc, cnts.at[c])


def _sc_pool(h3, bat2, zs, zc, ones):
    return pl.kernel(
        _sc_pool_body,
        out_type=[
            jax.ShapeDtypeStruct((2, _GP, 64), jnp.float32),
            jax.ShapeDtypeStruct((2, _GP, 16), jnp.float32),
        ],
        mesh=_mesh(),
        compiler_params=_SC_PARAMS,
        scratch_types=[
            pltpu.VMEM((_NR // 32, 128), jnp.int32),
            pltpu.VMEM((128, 64), jnp.float32),
            pltpu.VMEM((128, 16), jnp.float32),
            pltpu.VMEM_SHARED((_GP, 64), jnp.float32),
            pltpu.VMEM_SHARED((_GP, 16), jnp.float32),
            pltpu.SemaphoreType.DMA,
        ],
    )(h3, bat2, zs, zc, ones)


_R = 4096


def _dis_block(degp):
    deg = 1.0 + degp[0, :, :1] + degp[1, :, :1]
    return lax.rsqrt(deg)


def _tc_prep_body(degp, x, w1, hw_o, hs_o):
    dis = _dis_block(degp)
    hw = jnp.dot(x[...], w1[...], preferred_element_type=jnp.float32)
    hs = hw * dis
    hw_o[...] = hw
    for q in range(4):
        hs_o[q] = hs[:, 16 * q:16 * (q + 1)]


def _tc_prep(degp, x, w1):
    return pl.pallas_call(
        _tc_prep_body,
        grid=(_NP // _R,),
        in_specs=[
            pl.BlockSpec((2, _R, 16), lambda i: (0, i, 0)),
            pl.BlockSpec((_R, 64), lambda i: (i, 0)),
            pl.BlockSpec((64, 64), lambda i: (0, 0)),
        ],
        out_specs=[
            pl.BlockSpec((_R, 64), lambda i: (i, 0)),
            pl.BlockSpec((4, _R, 16), lambda i: (0, i, 0)),
        ],
        out_shape=[
            jax.ShapeDtypeStruct((_NP, 64), jnp.float32),
            jax.ShapeDtypeStruct((4, _NP, 16), jnp.float32),
        ],
    )(degp, x, w1)


def _tc_mid_body(degp, acc, hw, b, w, hw_o, hs_o):
    dis = _dis_block(degp)
    h = jnp.maximum(dis * acc[...] + (dis * dis) * hw[...] + b[...], 0.0)
    hw2 = jnp.dot(h, w[...], preferred_element_type=jnp.float32)
    hs2 = hw2 * dis
    hw_o[...] = hw2
    for q in range(4):
        hs_o[q] = hs2[:, 16 * q:16 * (q + 1)]


def _tc_mid(degp, acc, hw, b, w):
    return pl.pallas_call(
        _tc_mid_body,
        grid=(_NP // _R,),
        in_specs=[
            pl.BlockSpec((2, _R, 16), lambda i: (0, i, 0)),
            pl.BlockSpec((_R, 64), lambda i: (i, 0)),
            pl.BlockSpec((_R, 64), lambda i: (i, 0)),
            pl.BlockSpec((1, 64), lambda i: (0, 0)),
            pl.BlockSpec((64, 64), lambda i: (0, 0)),
        ],
        out_specs=[
            pl.BlockSpec((_R, 64), lambda i: (i, 0)),
            pl.BlockSpec((4, _R, 16), lambda i: (0, i, 0)),
        ],
        out_shape=[
            jax.ShapeDtypeStruct((_NP, 64), jnp.float32),
            jax.ShapeDtypeStruct((4, _NP, 16), jnp.float32),
        ],
    )(degp, acc, hw, b, w)


def _tc_last_body(degp, acc, hw, b, h_o):
    dis = _dis_block(degp)
    h_o[...] = jnp.maximum(dis * acc[...] + (dis * dis) * hw[...] + b[...], 0.0)


def _tc_last(degp, acc, hw, b):
    return pl.pallas_call(
        _tc_last_body,
        grid=(_NP // _R,),
        in_specs=[
            pl.BlockSpec((2, _R, 16), lambda i: (0, i, 0)),
            pl.BlockSpec((_R, 64), lambda i: (i, 0)),
            pl.BlockSpec((_R, 64), lambda i: (i, 0)),
            pl.BlockSpec((1, 64), lambda i: (0, 0)),
        ],
        out_specs=pl.BlockSpec((_R, 64), lambda i: (i, 0)),
        out_shape=jax.ShapeDtypeStruct((_NP, 64), jnp.float32),
    )(degp, acc, hw, b)


def _tc_head_body(sums, cnts, wc1, bc1, wc2, bc2, o):
    seg = sums[0, :_G, :] + sums[1, :_G, :]
    cnt = cnts[0, :_G, :1] + cnts[1, :_G, :1]
    pooled = seg / jnp.maximum(cnt, 1.0)
    z = jnp.maximum(
        jnp.dot(pooled, wc1[...], preferred_element_type=jnp.float32) + bc1[...],
        0.0,
    )
    o[...] = jnp.dot(z, wc2[...], preferred_element_type=jnp.float32) + bc2[...]


def _tc_head(sums, cnts, wc1, bc1, wc2, bc2):
    return pl.pallas_call(
        _tc_head_body,
        out_shape=jax.ShapeDtypeStruct((_G, 1), jnp.float32),
    )(sums, cnts, wc1, bc1, wc2, bc2)


def kernel(x, edge_index, batch, W1, b1, W2, b2, W3, b3, Wc1, bc1, Wc2, bc2):
    f32 = jnp.float32
    i32 = jnp.int32

    src = edge_index[0]
    dst = edge_index[1]
    pad_e = _EP - _E
    src2 = jnp.concatenate([src, jnp.zeros((pad_e,), i32)]).reshape(_ER, 128)
    dst2 = jnp.concatenate([dst, jnp.full((pad_e,), _N, i32)]).reshape(_ER, 128)
    bat2 = jnp.concatenate(
        [batch, jnp.full((_NP - _N,), _G, i32)]
    ).reshape(_NR, 128)
    x_p = jnp.concatenate([x, jnp.zeros((_NP - _N, 64), f32)], axis=0)

    ones16 = jnp.ones((128, 16), f32)
    z16 = jnp.zeros((_STRIPE, 16), f32)
    zgs = jnp.zeros((_GP, 64), f32)
    zgc = jnp.zeros((_GP, 16), f32)

    degp = _sc_deg(dst2, ones16, z16)

    hw1, hs1 = _tc_prep(degp, x_p, W1)
    acc1 = _sc_layer(hs1, src2, dst2, z16)
    hw2, hs2 = _tc_mid(degp, acc1, hw1, b1.reshape(1, 64), W2)
    acc2 = _sc_layer(hs2, src2, dst2, z16)
    hw3, hs3 = _tc_mid(degp, acc2, hw2, b2.reshape(1, 64), W3)
    acc3 = _sc_layer(hs3, src2, dst2, z16)
    h3 = _tc_last(degp, acc3, hw3, b3.reshape(1, 64))

    sums, cnts = _sc_pool(h3, bat2, zgs, zgc, ones16)
    return _tc_head(
        sums, cnts, Wc1, bc1.reshape(1, 32), Wc2, bc2.reshape(1, 1)
    )


# speedup vs baseline: 22.5481x; 1.1041x over previous
import functools

import jax
import jax.numpy as jnp
from jax import lax
from jax.experimental import pallas as pl
from jax.experimental.pallas import tpu as pltpu
from jax.experimental.pallas import tpu_sc as plsc

_N = 50000
_E = 800000
_G = 512
_H = 64

_NP = 53248
_EP = 802816
_ER = _EP // 128
_NR = _NP // 128
_GP = 520
_STRIPE = _NP // 16


def _mesh():
    return plsc.VectorSubcoreMesh(
        core_axis_name="c", subcore_axis_name="s", num_cores=2, num_subcores=16
    )


_SC_PARAMS = pltpu.CompilerParams(use_tc_tiling_on_sc=False)


def _sc_deg_body(dst2, ones, zeros, out, idx, ones_v, acc, sem):
    c = lax.axis_index("c")
    s = lax.axis_index("s")
    pltpu.sync_copy(zeros, acc.at[pl.ds(s * _STRIPE, _STRIPE)])
    pltpu.sync_copy(ones, ones_v)
    plsc.subcore_barrier()
    row0 = (c * 16 + s) * (_ER // 32)
    nb, bw = 14, 14

    def load(b, buf):
        pltpu.sync_copy(dst2.at[pl.ds(row0 + b * bw, bw)], idx.at[buf])

    def fire(buf):
        for j in range(bw):
            pltpu.async_copy(ones_v, acc.at[idx.at[buf].at[j]], sem, add=True)

    def drain(buf):
        for j in range(bw):
            pltpu.make_async_copy(
                ones_v, acc.at[idx.at[buf].at[j]], sem
            ).wait()

    load(0, 0)
    @pl.loop(0, nb)
    def _blk(b):
        buf = lax.rem(b, 2)
        nbuf = lax.rem(b + 1, 2)
        @pl.when(b > 0)
        def _w():
            drain(nbuf)
        @pl.when(b < nb - 1)
        def _l():
            load(b + 1, nbuf)
        fire(buf)
    drain((nb - 1) % 2)
    plsc.subcore_barrier()
    pltpu.sync_copy(
        acc.at[pl.ds(s * _STRIPE, _STRIPE)],
        out.at[c].at[pl.ds(s * _STRIPE, _STRIPE)],
    )


def _sc_deg(dst2, ones, zeros):
    return pl.kernel(
        _sc_deg_body,
        out_type=jax.ShapeDtypeStruct((2, _NP, 16), jnp.float32),
        mesh=_mesh(),
        compiler_params=_SC_PARAMS,
        scratch_types=[
            pltpu.VMEM((2, 14, 128), jnp.int32),
            pltpu.VMEM((128, 16), jnp.float32),
            pltpu.VMEM_SHARED((_NP, 16), jnp.float32),
            pltpu.SemaphoreType.DMA,
        ],
    )(dst2, ones, zeros)


def _sc_layer_body(hs4, src4q, dst2, zeros, out, idx_s, idx_d, rows, acc, sem, sem2):
    c = lax.axis_index("c")
    s = lax.axis_index("s")
    row0 = s * (_ER // 16)
    nb, bw = 28, 14
    for p in range(2):
        q = 2 * c + p
        pltpu.sync_copy(zeros, acc.at[pl.ds(s * _STRIPE, _STRIPE)])
        plsc.subcore_barrier()
        table = hs4

        def load_fire(b, buf):
            r0 = row0 + b * bw
            pltpu.sync_copy(src4q.at[q].at[pl.ds(r0, bw)], idx_s.at[buf])
            pltpu.sync_copy(dst2.at[pl.ds(r0, bw)], idx_d.at[buf])
            for j in range(bw):
                pltpu.async_copy(
                    table.at[idx_s.at[buf].at[j]], rows.at[buf].at[j], sem
                )

        def wait_gathers(buf):
            for j in range(bw):
                pltpu.make_async_copy(
                    table.at[idx_s.at[buf].at[j]], rows.at[buf].at[j], sem
                ).wait()

        def fire_scatters(buf):
            for j in range(bw):
                pltpu.async_copy(
                    rows.at[buf].at[j], acc.at[idx_d.at[buf].at[j]], sem2, add=True
                )

        def wait_scatters(buf):
            for j in range(bw):
                pltpu.make_async_copy(
                    rows.at[buf].at[j], acc.at[idx_d.at[buf].at[j]], sem2
                ).wait()

        load_fire(0, 0)
        @pl.loop(0, nb)
        def _blk(b):
            buf = lax.rem(b, 2)
            nbuf = lax.rem(b + 1, 2)
            @pl.when(b > 0)
            def _ws():
                wait_scatters(nbuf)
            @pl.when(b < nb - 1)
            def _lf():
                load_fire(b + 1, nbuf)
            wait_gathers(buf)
            fire_scatters(buf)
        wait_scatters((nb - 1) % 2)
        plsc.subcore_barrier()
        pltpu.sync_copy(
            acc.at[pl.ds(s * _STRIPE, _STRIPE)],
            out.at[pl.ds(s * _STRIPE, _STRIPE), pl.ds(16 * q, 16)],
        )
        plsc.subcore_barrier()


def _sc_layer(hs, src4q, dst2, zeros):
    hs4 = hs.reshape(4 * _NP, 16)
    return pl.kernel(
        _sc_layer_body,
        out_type=jax.ShapeDtypeStruct((_NP, 64), jnp.float32),
        mesh=_mesh(),
        compiler_params=_SC_PARAMS,
        scratch_types=[
            pltpu.VMEM((2, 14, 128), jnp.int32),
            pltpu.VMEM((2, 14, 128), jnp.int32),
            pltpu.VMEM((2, 14, 128, 16), jnp.float32),
            pltpu.VMEM_SHARED((_NP, 16), jnp.float32),
            pltpu.SemaphoreType.DMA,
            pltpu.SemaphoreType.DMA,
        ],
    )(hs4, src4q, dst2, zeros)


def _sc_pool_body(h3, bat2, zs, zc, ones, sums, cnts,
                  idx, rows_v, ones_v, segacc, cntacc, sem):
    c = lax.axis_index("c")
    s = lax.axis_index("s")

    @pl.when(s == 0)
    def _zero():
        pltpu.sync_copy(zs, segacc)
        pltpu.sync_copy(zc, cntacc)

    pltpu.sync_copy(ones, ones_v)
    rpt = _NR // 32
    row0 = c * (_NR // 2) + s * rpt
    pltpu.sync_copy(bat2.at[pl.ds(row0, rpt)], idx)
    plsc.subcore_barrier()
    @pl.loop(0, rpt)
    def _blk(b):
        pltpu.sync_copy(h3.at[pl.ds((row0 + b) * 128, 128)], rows_v)
        pltpu.sync_copy(rows_v, segacc.at[idx.at[b]], add=True)
        pltpu.sync_copy(ones_v, cntacc.at[idx.at[b]], add=True)
    plsc.subcore_barrier()

    @pl.when(s == 0)
    def _drain():
        pltpu.sync_copy(segacc, sums.at[c])
        pltpu.sync_copy(cntacc, cnts.at[c])


def _sc_pool(h3, bat2, zs, zc, ones):
    return pl.kernel(
        _sc_pool_body,
        out_type=[
            jax.ShapeDtypeStruct((2, _GP, 64), jnp.float32),
            jax.ShapeDtypeStruct((2, _GP, 16), jnp.float32),
        ],
        mesh=_mesh(),
        compiler_params=_SC_PARAMS,
        scratch_types=[
            pltpu.VMEM((_NR // 32, 128), jnp.int32),
            pltpu.VMEM((128, 64), jnp.float32),
            pltpu.VMEM((128, 16), jnp.float32),
            pltpu.VMEM_SHARED((_GP, 64), jnp.float32),
            pltpu.VMEM_SHARED((_GP, 16), jnp.float32),
            pltpu.SemaphoreType.DMA,
        ],
    )(h3, bat2, zs, zc, ones)


_R = 4096


def _dis_block(degp):
    deg = 1.0 + degp[0, :, :1] + degp[1, :, :1]
    return lax.rsqrt(deg)


def _tc_prep_body(degp, x, w1, hw_o, hs_o):
    dis = _dis_block(degp)
    hw = jnp.dot(x[...], w1[...], preferred_element_type=jnp.float32)
    hw_o[...] = hw
    hs_o[...] = hw * dis


def _tc_prep(degp, x, w1):
    return pl.pallas_call(
        _tc_prep_body,
        grid=(_NP // _R,),
        in_specs=[
            pl.BlockSpec((2, _R, 16), lambda i: (0, i, 0)),
            pl.BlockSpec((_R, 64), lambda i: (i, 0)),
            pl.BlockSpec((64, 64), lambda i: (0, 0)),
        ],
        out_specs=[
            pl.BlockSpec((_R, 64), lambda i: (i, 0)),
            pl.BlockSpec((_R, 64), lambda i: (i, 0)),
        ],
        out_shape=[
            jax.ShapeDtypeStruct((_NP, 64), jnp.float32),
            jax.ShapeDtypeStruct((_NP, 64), jnp.float32),
        ],
    )(degp, x, w1)


def _tc_mid_body(degp, acc, hw, b, w, hw_o, hs_o):
    dis = _dis_block(degp)
    h = jnp.maximum(dis * acc[...] + (dis * dis) * hw[...] + b[...], 0.0)
    hw2 = jnp.dot(h, w[...], preferred_element_type=jnp.float32)
    hw_o[...] = hw2
    hs_o[...] = hw2 * dis


def _tc_mid(degp, acc, hw, b, w):
    return pl.pallas_call(
        _tc_mid_body,
        grid=(_NP // _R,),
        in_specs=[
            pl.BlockSpec((2, _R, 16), lambda i: (0, i, 0)),
            pl.BlockSpec((_R, 64), lambda i: (i, 0)),
            pl.BlockSpec((_R, 64), lambda i: (i, 0)),
            pl.BlockSpec((1, 64), lambda i: (0, 0)),
            pl.BlockSpec((64, 64), lambda i: (0, 0)),
        ],
        out_specs=[
            pl.BlockSpec((_R, 64), lambda i: (i, 0)),
            pl.BlockSpec((_R, 64), lambda i: (i, 0)),
        ],
        out_shape=[
            jax.ShapeDtypeStruct((_NP, 64), jnp.float32),
            jax.ShapeDtypeStruct((_NP, 64), jnp.float32),
        ],
    )(degp, acc, hw, b, w)


def _tc_last_body(degp, acc, hw, b, h_o):
    dis = _dis_block(degp)
    h_o[...] = jnp.maximum(dis * acc[...] + (dis * dis) * hw[...] + b[...], 0.0)


def _tc_last(degp, acc, hw, b):
    return pl.pallas_call(
        _tc_last_body,
        grid=(_NP // _R,),
        in_specs=[
            pl.BlockSpec((2, _R, 16), lambda i: (0, i, 0)),
            pl.BlockSpec((_R, 64), lambda i: (i, 0)),
            pl.BlockSpec((_R, 64), lambda i: (i, 0)),
            pl.BlockSpec((1, 64), lambda i: (0, 0)),
        ],
        out_specs=pl.BlockSpec((_R, 64), lambda i: (i, 0)),
        out_shape=jax.ShapeDtypeStruct((_NP, 64), jnp.float32),
    )(degp, acc, hw, b)


def _tc_head_body(sums, cnts, wc1, bc1, wc2, bc2, o):
    seg = sums[0, :_G, :] + sums[1, :_G, :]
    cnt = cnts[0, :_G, :1] + cnts[1, :_G, :1]
    pooled = seg / jnp.maximum(cnt, 1.0)
    z = jnp.maximum(
        jnp.dot(pooled, wc1[...], preferred_element_type=jnp.float32) + bc1[...],
        0.0,
    )
    o[...] = jnp.dot(z, wc2[...], preferred_element_type=jnp.float32) + bc2[...]


def _tc_head(sums, cnts, wc1, bc1, wc2, bc2):
    return pl.pallas_call(
        _tc_head_body,
        out_shape=jax.ShapeDtypeStruct((_G, 1), jnp.float32),
    )(sums, cnts, wc1, bc1, wc2, bc2)


def kernel(x, edge_index, batch, W1, b1, W2, b2, W3, b3, Wc1, bc1, Wc2, bc2):
    f32 = jnp.float32
    i32 = jnp.int32

    src = edge_index[0]
    dst = edge_index[1]
    pad_e = _EP - _E
    src2 = jnp.concatenate([src, jnp.zeros((pad_e,), i32)]).reshape(_ER, 128)
    dst2 = jnp.concatenate([dst, jnp.full((pad_e,), _N, i32)]).reshape(_ER, 128)
    src4q = 4 * src2[None] + jnp.arange(4, dtype=i32)[:, None, None]
    bat2 = jnp.concatenate(
        [batch, jnp.full((_NP - _N,), _G, i32)]
    ).reshape(_NR, 128)
    x_p = jnp.concatenate([x, jnp.zeros((_NP - _N, 64), f32)], axis=0)

    ones16 = jnp.ones((128, 16), f32)
    z16 = jnp.zeros((_STRIPE, 16), f32)
    zgs = jnp.zeros((_GP, 64), f32)
    zgc = jnp.zeros((_GP, 16), f32)

    degp = _sc_deg(dst2, ones16, z16)

    hw1, hs1 = _tc_prep(degp, x_p, W1)
    acc1 = _sc_layer(hs1, src4q, dst2, z16)
    hw2, hs2 = _tc_mid(degp, acc1, hw1, b1.reshape(1, 64), W2)
    acc2 = _sc_layer(hs2, src4q, dst2, z16)
    hw3, hs3 = _tc_mid(degp, acc2, hw2, b2.reshape(1, 64), W3)
    acc3 = _sc_layer(hs3, src4q, dst2, z16)
    h3 = _tc_last(degp, acc3, hw3, b3.reshape(1, 64))

    sums, cnts = _sc_pool(h3, bat2, zgs, zgc, ones16)
    return _tc_head(
        sums, cnts, Wc1, bc1.reshape(1, 32), Wc2, bc2.reshape(1, 1)
    )
